# Initial kernel scaffold; baseline (speedup 1.0000x reference)
import jax
import jax.numpy as jnp
from jax.experimental import pallas as pl


def kernel(x, edge_index, edge_attr, Wl1, bl1, Wr1, br1, We1, att1, b1, Wl2, bl2, Wr2, br2, We2, att2, b2, D1, d1b, D2, d2b, W_fc1, b_fc1, W_fc2, b_fc2, W_mean, b_mean, W_lstd, b_lstd):
    raise NotImplementedError("write your pallas kernel here")


# speedup vs baseline: 1.1128x; 1.1128x over previous
import jax
import jax.numpy as jnp
from jax.experimental import pallas as pl
from jax.experimental.pallas import tpu as pltpu

_LOG_STD_MAX, _LOG_STD_MIN = 2.0, -5.0


def _mm_bias(x, W, b, bm=2000):
    M, K = x.shape
    Nc = W.shape[1]
    if M % bm != 0:
        bm = M
    b2 = b.reshape(1, Nc)

    def body(x_ref, w_ref, b_ref, o_ref):
        o_ref[...] = (
            jnp.dot(x_ref[...], w_ref[...], preferred_element_type=jnp.float32)
            + b_ref[...]
        )

    return pl.pallas_call(
        body,
        grid=(M // bm,),
        in_specs=[
            pl.BlockSpec((bm, K), lambda i: (i, 0)),
            pl.BlockSpec((K, Nc), lambda i: (0, 0)),
            pl.BlockSpec((1, Nc), lambda i: (0, 0)),
        ],
        out_specs=pl.BlockSpec((bm, Nc), lambda i: (i, 0)),
        out_shape=jax.ShapeDtypeStruct((M, Nc), jnp.float32),
    )(x, W, b2)


def _head(ego, D1, d1b, D2, d2b, W_fc1, b_fc1, W_fc2, b_fc2,
          W_mean, b_mean, W_lstd, b_lstd):
    ego2 = ego.reshape(1, -1)

    def body(e_ref, D1r, d1br, D2r, d2br, f1r, fb1r, f2r, fb2r,
             wmr, bmr, wlr, blr, mean_ref, lstd_ref):
        t = jnp.dot(e_ref[...], D1r[...], preferred_element_type=jnp.float32) + d1br[...]
        z = jnp.tanh(jnp.dot(t, D2r[...], preferred_element_type=jnp.float32) + d2br[...])
        a = jax.nn.relu(jnp.dot(z, f1r[...], preferred_element_type=jnp.float32) + fb1r[...])
        a = jax.nn.relu(jnp.dot(a, f2r[...], preferred_element_type=jnp.float32) + fb2r[...])
        mean_ref[...] = jnp.dot(a, wmr[...], preferred_element_type=jnp.float32) + bmr[...]
        l = jnp.tanh(jnp.dot(a, wlr[...], preferred_element_type=jnp.float32) + blr[...])
        lstd_ref[...] = _LOG_STD_MIN + 0.5 * (_LOG_STD_MAX - _LOG_STD_MIN) * (l + 1.0)

    mean, lstd = pl.pallas_call(
        body,
        out_shape=(
            jax.ShapeDtypeStruct((1, 2), jnp.float32),
            jax.ShapeDtypeStruct((1, 2), jnp.float32),
        ),
    )(ego2, D1, d1b.reshape(1, -1), D2, d2b.reshape(1, -1),
      W_fc1, b_fc1.reshape(1, -1), W_fc2, b_fc2.reshape(1, -1),
      W_mean, b_mean.reshape(1, -1), W_lstd, b_lstd.reshape(1, -1))
    return mean.reshape(-1), lstd.reshape(-1)


def kernel(x, edge_index, edge_attr, Wl1, bl1, Wr1, br1, We1, att1, b1,
           Wl2, bl2, Wr2, br2, We2, att2, b2, D1, d1b, D2, d2b,
           W_fc1, b_fc1, W_fc2, b_fc2, W_mean, b_mean, W_lstd, b_lstd):
    src, dst = edge_index[0], edge_index[1]
    n = x.shape[0]
    H1, C1 = att1.shape
    C2 = att2.shape[1]

    xl1 = _mm_bias(x, Wl1, bl1).reshape(n, H1, C1)
    xr1 = _mm_bias(x, Wr1, br1).reshape(n, H1, C1)
    ee1 = _mm_bias(edge_attr, We1, jnp.zeros_like(bl1)).reshape(-1, H1, C1)

    m = jax.nn.leaky_relu(xl1[src] + xr1[dst] + ee1, 0.2)
    e = jnp.sum(m * att1[None, :, :], axis=-1)
    emax = jax.ops.segment_max(e, dst, num_segments=n)
    ex = jnp.exp(e - emax[dst])
    denom = jax.ops.segment_sum(ex, dst, num_segments=n)
    alpha = ex / (denom[dst] + 1e-16)
    out1 = jax.ops.segment_sum(xl1[src] * alpha[:, :, None], dst, num_segments=n)
    h1 = jax.nn.relu(out1.reshape(n, H1 * C1) + b1)

    xl2 = _mm_bias(h1, Wl2, bl2)
    xr2_0 = _mm_bias(h1[0:1], Wr2, br2)
    ee2 = _mm_bias(edge_attr, We2, jnp.zeros_like(bl2))

    rows = xl2[src]
    m2 = jax.nn.leaky_relu(rows + xr2_0 + ee2, 0.2)
    e2 = jnp.sum(m2 * att2[0][None, :], axis=-1)
    mask = dst == 0
    e2m = jnp.where(mask, e2, -1e30)
    kmax = jnp.maximum(jnp.max(e2m), -1e30)
    ex2 = jnp.where(mask, jnp.exp(e2m - kmax), 0.0)
    denom2 = jnp.sum(ex2)
    ego = jax.nn.relu(ex2 @ rows / (denom2 + 1e-16) + b2)

    return _head(ego, D1, d1b, D2, d2b, W_fc1, b_fc1, W_fc2, b_fc2,
                 W_mean, b_mean, W_lstd, b_lstd)


# speedup vs baseline: 5.3076x; 4.7695x over previous
import functools

import jax
import jax.numpy as jnp
from jax import lax
from jax.experimental import pallas as pl
from jax.experimental.pallas import tpu as pltpu
from jax.experimental.pallas import tpu_sc as plsc

_LOG_STD_MAX, _LOG_STD_MIN = 2.0, -5.0

_N = 10000
_E = 80000
_B = 64
_NCHUNK = _E // _B
_NW = 32

_mesh = plsc.VectorSubcoreMesh(core_axis_name="c", subcore_axis_name="s")


def _mm_bias(x, W, b, bm=2000):
    M, K = x.shape
    Nc = W.shape[1]
    if M % bm != 0:
        bm = M
    b2 = b.reshape(1, Nc)

    def body(x_ref, w_ref, b_ref, o_ref):
        o_ref[...] = (
            jnp.dot(x_ref[...], w_ref[...], preferred_element_type=jnp.float32)
            + b_ref[...]
        )

    return pl.pallas_call(
        body,
        grid=(M // bm,),
        in_specs=[
            pl.BlockSpec((bm, K), lambda i: (i, 0)),
            pl.BlockSpec((K, Nc), lambda i: (0, 0)),
            pl.BlockSpec((1, Nc), lambda i: (0, 0)),
        ],
        out_specs=pl.BlockSpec((bm, Nc), lambda i: (i, 0)),
        out_shape=jax.ShapeDtypeStruct((M, Nc), jnp.float32),
    )(x, W, b2)


def _sc_logits(xls, xr1, ee1, src, dst, att1):
    @functools.partial(
        pl.kernel,
        mesh=_mesh,
        compiler_params=pltpu.CompilerParams(use_tc_tiling_on_sc=False),
        out_type=jax.ShapeDtypeStruct((_E, 80), jnp.float32),
        scratch_types=[
            pltpu.VMEM((_B,), jnp.int32),
            pltpu.VMEM((_B,), jnp.int32),
            pltpu.VMEM((_B, 80), jnp.float32),
            pltpu.VMEM((_B, 80), jnp.float32),
            pltpu.VMEM((_B, 80), jnp.float32),
            pltpu.VMEM((_B, 80), jnp.float32),
            pltpu.VMEM((_B, 80), jnp.float32),
            pltpu.VMEM((_B, 400), jnp.float32),
            pltpu.VMEM((_B, 400), jnp.float32),
            pltpu.VMEM((_B, 80), jnp.float32),
            pltpu.VMEM((5, 80), jnp.float32),
            pltpu.SemaphoreType.DMA,
        ],
    )
    def k(xl0_h, xl1_h, xl2_h, xl3_h, xl4_h, xr_h, ee_h, src_h, dst_h,
          att_h, acc_h,
          src_v, dst_v, r0, r1, r2, r3, r4, rr, re, acc_v, att_v, sem):
        xlt = [xl0_h, xl1_h, xl2_h, xl3_h, xl4_h]
        rv = [r0, r1, r2, r3, r4]
        wid = lax.axis_index("s") * 2 + lax.axis_index("c")
        pltpu.sync_copy(att_h, att_v)

        nc = (_NCHUNK - 1 - wid) // _NW + 1

        def chunk_body(j, carry):
            base = (wid + j * _NW) * _B
            pltpu.sync_copy(src_h.at[pl.ds(base, _B)], src_v)
            pltpu.sync_copy(dst_h.at[pl.ds(base, _B)], dst_v)
            for t in range(5):
                pltpu.async_copy(xlt[t].at[src_v], rv[t], sem).wait()
            pltpu.async_copy(xr_h.at[dst_v], rr, sem).wait()
            pltpu.sync_copy(ee_h.at[pl.ds(base, _B), :], re)

            def edge_body(i, c2):
                for h in range(5):
                    acc = jnp.zeros((16,), jnp.float32)
                    for v in range(5):
                        g = h * 80 + v * 16
                        xl = rv[h][i, pl.ds(v * 16, 16)]
                        s = xl + rr[i, pl.ds(g, 16)] + re[i, pl.ds(g, 16)]
                        m = jnp.maximum(s, 0.2 * s)
                        acc = acc + m * att_v[h, pl.ds(v * 16, 16)]
                    acc_v[i, pl.ds(h * 16, 16)] = acc
                return c2

            lax.fori_loop(0, _B, edge_body, 0)
            pltpu.sync_copy(acc_v, acc_h.at[pl.ds(base, _B), :])
            return carry

        lax.fori_loop(0, nc, chunk_body, 0)

    return k(xls[0], xls[1], xls[2], xls[3], xls[4], xr1, ee1, src, dst, att1)


def _head_max(acc, bm=8000):
    def body(a_ref, o_ref):
        i = pl.program_id(0)
        c = jax.lax.broadcasted_iota(jnp.int32, (80, 5), 0) // 16
        hh = jax.lax.broadcasted_iota(jnp.int32, (80, 5), 1)
        S = (c == hh).astype(jnp.float32)
        e = jnp.dot(a_ref[...], S, preferred_element_type=jnp.float32)
        part = jnp.max(e, axis=0).reshape(5, 1)
        part = jnp.broadcast_to(part, (5, 128))

        @pl.when(i == 0)
        def _():
            o_ref[...] = part

        @pl.when(i != 0)
        def _():
            o_ref[...] = jnp.maximum(o_ref[...], part)

    return pl.pallas_call(
        body,
        grid=(_E // bm,),
        in_specs=[pl.BlockSpec((bm, 80), lambda i: (i, 0))],
        out_specs=pl.BlockSpec((5, 128), lambda i: (0, 0)),
        out_shape=jax.ShapeDtypeStruct((5, 128), jnp.float32),
    )(acc)


def _expsplat(acc, kmax, bm=8000):
    def body(a_ref, k_ref, o_ref):
        c = jax.lax.broadcasted_iota(jnp.int32, (80, 5), 0) // 16
        hh = jax.lax.broadcasted_iota(jnp.int32, (80, 5), 1)
        S = (c == hh).astype(jnp.float32)
        cT = jax.lax.broadcasted_iota(jnp.int32, (5, 96), 1) // 16
        hT = jax.lax.broadcasted_iota(jnp.int32, (5, 96), 0)
        ST = (cT == hT).astype(jnp.float32)
        e = jnp.dot(a_ref[...], S, preferred_element_type=jnp.float32)
        ex = jnp.exp(e - k_ref[:, 0][None, :])
        o_ref[...] = jnp.dot(ex, ST, preferred_element_type=jnp.float32)

    return pl.pallas_call(
        body,
        grid=(_E // bm,),
        in_specs=[
            pl.BlockSpec((bm, 80), lambda i: (i, 0)),
            pl.BlockSpec((5, 128), lambda i: (0, 0)),
        ],
        out_specs=pl.BlockSpec((bm, 96), lambda i: (i, 0)),
        out_shape=jax.ShapeDtypeStruct((_E, 96), jnp.float32),
    )(acc, kmax)


def _sc_aggregate(xl_head, exb, src, dst, head):
    width = 96

    @functools.partial(
        pl.kernel,
        mesh=_mesh,
        compiler_params=pltpu.CompilerParams(use_tc_tiling_on_sc=False),
        out_type=jax.ShapeDtypeStruct((2, _N, width), jnp.float32),
        scratch_types=[
            pltpu.VMEM((_B,), jnp.int32),
            pltpu.VMEM((_B,), jnp.int32),
            pltpu.VMEM((_B, 80), jnp.float32),
            pltpu.VMEM((_B, 96), jnp.float32),
            pltpu.VMEM((_B, width), jnp.float32),
            pltpu.VMEM((16, width), jnp.float32),
            pltpu.VMEM_SHARED((_N, width), jnp.float32),
            pltpu.SemaphoreType.DMA,
        ],
    )
    def k(xl_h, exb_h, src_h, dst_h, out_h,
          src_v, dst_v, rows_v, exb_v, w_v, z_v, acc_s, sem):
        cid = lax.axis_index("c")
        sid = lax.axis_index("s")
        wid = sid * 2 + cid

        for r in range(16):
            for v in range(width // 16):
                z_v[r, pl.ds(v * 16, 16)] = jnp.zeros((16,), jnp.float32)
        rows_per_sub = _N // 16
        nz = rows_per_sub // 16
        for r in range(nz):
            pltpu.sync_copy(z_v, acc_s.at[pl.ds(sid * rows_per_sub + r * 16, 16), :])
        pltpu.sync_copy(
            z_v.at[pl.ds(0, rows_per_sub - nz * 16), :],
            acc_s.at[pl.ds(sid * rows_per_sub + nz * 16, rows_per_sub - nz * 16), :],
        )
        plsc.subcore_barrier()

        nc = (_NCHUNK - 1 - wid) // _NW + 1
        lane = lax.iota(jnp.int32, 16)

        def chunk_body(j, carry):
            base = (wid + j * _NW) * _B
            pltpu.sync_copy(src_h.at[pl.ds(base, _B)], src_v)
            pltpu.sync_copy(dst_h.at[pl.ds(base, _B)], dst_v)
            pltpu.async_copy(xl_h.at[src_v], rows_v, sem).wait()
            pltpu.sync_copy(exb_h.at[pl.ds(base, _B), :], exb_v)

            def edge_body(i, c2):
                exbh = exb_v[i, pl.ds(head * 16, 16)]
                for v in range(5):
                    w_v[i, pl.ds(v * 16, 16)] = rows_v[i, pl.ds(v * 16, 16)] * exbh
                tail = jnp.where(lane == 0, exbh, jnp.zeros((16,), jnp.float32))
                w_v[i, pl.ds(80, 16)] = tail
                return c2

            lax.fori_loop(0, _B, edge_body, 0)
            pltpu.sync_copy(w_v, acc_s.at[dst_v], add=True)
            return carry

        lax.fori_loop(0, nc, chunk_body, 0)
        plsc.subcore_barrier()

        for r in range(rows_per_sub // 16 + 1):
            r0 = sid * rows_per_sub + r * 16
            n = min(16, rows_per_sub - r * 16)
            if n > 0:
                pltpu.sync_copy(
                    acc_s.at[pl.ds(r0, n), :], out_h.at[cid, pl.ds(r0, n), :])

    return k(xl_head, exb, src, dst)


def _head(ego, D1, d1b, D2, d2b, W_fc1, b_fc1, W_fc2, b_fc2,
          W_mean, b_mean, W_lstd, b_lstd):
    ego2 = ego.reshape(1, -1)

    def body(e_ref, D1r, d1br, D2r, d2br, f1r, fb1r, f2r, fb2r,
             wmr, bmr, wlr, blr, mean_ref, lstd_ref):
        t = jnp.dot(e_ref[...], D1r[...], preferred_element_type=jnp.float32) + d1br[...]
        z = jnp.tanh(jnp.dot(t, D2r[...], preferred_element_type=jnp.float32) + d2br[...])
        a = jax.nn.relu(jnp.dot(z, f1r[...], preferred_element_type=jnp.float32) + fb1r[...])
        a = jax.nn.relu(jnp.dot(a, f2r[...], preferred_element_type=jnp.float32) + fb2r[...])
        mean_ref[...] = jnp.dot(a, wmr[...], preferred_element_type=jnp.float32) + bmr[...]
        l = jnp.tanh(jnp.dot(a, wlr[...], preferred_element_type=jnp.float32) + blr[...])
        lstd_ref[...] = _LOG_STD_MIN + 0.5 * (_LOG_STD_MAX - _LOG_STD_MIN) * (l + 1.0)

    mean, lstd = pl.pallas_call(
        body,
        out_shape=(
            jax.ShapeDtypeStruct((1, 2), jnp.float32),
            jax.ShapeDtypeStruct((1, 2), jnp.float32),
        ),
    )(ego2, D1, d1b.reshape(1, -1), D2, d2b.reshape(1, -1),
      W_fc1, b_fc1.reshape(1, -1), W_fc2, b_fc2.reshape(1, -1),
      W_mean, b_mean.reshape(1, -1), W_lstd, b_lstd.reshape(1, -1))
    return mean.reshape(-1), lstd.reshape(-1)


def _combine_h1(ps, b1, bn=2000):
    def body(p0r, p1r, p2r, p3r, p4r, br, o_ref):
        outs = []
        for pr in (p0r, p1r, p2r, p3r, p4r):
            s = pr[0] + pr[1]
            den = s[:, 80][:, None]
            den = jnp.where(den > 0.0, den, 1.0)
            outs.append(s[:, :80] / den)
        o_ref[...] = jax.nn.relu(jnp.concatenate(outs, axis=1) + br[...])

    return pl.pallas_call(
        body,
        grid=(_N // bn,),
        in_specs=[pl.BlockSpec((2, bn, 96), lambda i: (0, i, 0))] * 5
        + [pl.BlockSpec((1, 400), lambda i: (0, 0))],
        out_specs=pl.BlockSpec((bn, 400), lambda i: (i, 0)),
        out_shape=jax.ShapeDtypeStruct((_N, 400), jnp.float32),
    )(*ps, b1.reshape(1, 400))


def kernel(x, edge_index, edge_attr, Wl1, bl1, Wr1, br1, We1, att1, b1,
           Wl2, bl2, Wr2, br2, We2, att2, b2, D1, d1b, D2, d2b,
           W_fc1, b_fc1, W_fc2, b_fc2, W_mean, b_mean, W_lstd, b_lstd):
    src, dst = edge_index[0], edge_index[1]
    C2 = att2.shape[1]
    z400 = jnp.zeros((400,), jnp.float32)

    xls = [_mm_bias(x, Wl1[:, 80 * h:80 * (h + 1)], bl1[80 * h:80 * (h + 1)])
           for h in range(5)]
    xr1 = _mm_bias(x, Wr1, br1)
    ee1 = _mm_bias(edge_attr, We1, z400)

    acc = _sc_logits(xls, xr1, ee1, src, dst, att1)
    kmax = _head_max(acc)
    exb = _expsplat(acc, kmax)
    ps = [_sc_aggregate(xls[h], exb, src, dst, h) for h in range(5)]
    h1 = _combine_h1(ps, b1)

    xl2 = _mm_bias(h1, Wl2, bl2)
    xr2_0 = _mm_bias(h1[0:1], Wr2, br2)
    ee2 = _mm_bias(edge_attr, We2, jnp.zeros((C2,), jnp.float32))

    rows = xl2[src]
    m2 = jax.nn.leaky_relu(rows + xr2_0 + ee2, 0.2)
    e2 = jnp.sum(m2 * att2[0][None, :], axis=-1)
    mask = dst == 0
    e2m = jnp.where(mask, e2, -1e30)
    km2 = jnp.maximum(jnp.max(e2m), -1e30)
    ex2 = jnp.where(mask, jnp.exp(e2m - km2), 0.0)
    denom2 = jnp.sum(ex2)
    ego = jax.nn.relu(ex2 @ rows / (denom2 + 1e-16) + b2)

    return _head(ego, D1, d1b, D2, d2b, W_fc1, b_fc1, W_fc2, b_fc2,
                 W_mean, b_mean, W_lstd, b_lstd)
